# baseline (speedup 1.0000x reference)
import functools

import jax
import jax.numpy as jnp
from jax import lax
from jax.experimental import pallas as pl
from jax.experimental.pallas import tpu as pltpu
from jax.experimental.pallas import tpu_sc as plsc

D = 64
T = 200
HALF = T // 2
NC = 2
NS = 16
NW = NC * NS
C = 128
NBUF = 4
G = 8


def _sc_gather(flat_ids, table):
    n_flat = flat_ids.shape[0]
    per_w = n_flat // NW
    n_chunks = per_w // C

    mesh = plsc.VectorSubcoreMesh(core_axis_name="c", subcore_axis_name="s")

    @functools.partial(
        pl.kernel,
        mesh=mesh,
        compiler_params=pltpu.CompilerParams(use_tc_tiling_on_sc=False),
        out_type=jax.ShapeDtypeStruct((n_flat, D), jnp.float32),
        scratch_types=[
            pltpu.VMEM((per_w,), jnp.int32),
        ] + [pltpu.VMEM((C, D), jnp.float32)] * NBUF
          + [pltpu.SemaphoreType.DMA] * (2 * NBUF),
    )
    def body(ids_hbm, table_hbm, out_hbm, idx_all,
             r0, r1, r2, r3, sg0, sg1, sg2, sg3, sw0, sw1, sw2, sw3):
        rows = (r0, r1, r2, r3)
        sg = (sg0, sg1, sg2, sg3)
        sw = (sw0, sw1, sw2, sw3)
        wid = lax.axis_index("s") * NC + lax.axis_index("c")
        base = wid * per_w
        pltpu.sync_copy(ids_hbm.at[pl.ds(base, per_w)], idx_all)

        def gather_start(g, b):
            pltpu.async_copy(table_hbm.at[idx_all.at[pl.ds(g * C, C)]],
                             rows[b], sg[b])

        def gather_wait(b):
            pltpu.make_async_copy(table_hbm.at[idx_all.at[pl.ds(0, C)]],
                                  rows[b], sg[b]).wait()

        def write_start(g, b):
            pltpu.async_copy(rows[b], out_hbm.at[pl.ds(base + g * C, C)],
                             sw[b])

        def write_wait(b):
            pltpu.make_async_copy(rows[b], out_hbm.at[pl.ds(base, C)],
                                  sw[b]).wait()

        for b in range(NBUF):
            gather_start(b, b)

        def main_body(i, carry):
            k = i * NBUF
            for b in range(NBUF):
                gather_wait(b)
                write_start(k + b, b)
            for b in range(NBUF):
                write_wait(b)
                gather_start(k + NBUF + b, b)
            return carry

        lax.fori_loop(0, n_chunks // NBUF - 1, main_body, 0)

        k = n_chunks - NBUF
        for b in range(NBUF):
            gather_wait(b)
            write_start(k + b, b)
        for b in range(NBUF):
            write_wait(b)

    return body(flat_ids, table)


def _tc_add(inter, pos_pair, batch):
    def tc_body(inter_ref, pos_ref, out_ref):
        x = inter_ref[...].reshape(G, HALF, 2 * D) + pos_ref[...][None]
        out_ref[:, :HALF, :] = x[:, :, :D]
        out_ref[:, HALF:, :] = x[:, :, D:]

    return pl.pallas_call(
        tc_body,
        grid=(batch // G,),
        in_specs=[
            pl.BlockSpec((G * HALF, 2 * D), lambda i: (i, 0)),
            pl.BlockSpec((HALF, 2 * D), lambda i: (0, 0)),
        ],
        out_specs=pl.BlockSpec((G, T, D), lambda i: (i, 0, 0)),
        out_shape=jax.ShapeDtypeStruct((batch, T, D), jnp.float32),
    )(inter, pos_pair)


def kernel(input_ids, phoneme_table, position_table):
    b, t = input_ids.shape
    flat_ids = (input_ids.astype(jnp.int32)
                .reshape(b, 2, HALF).transpose(0, 2, 1).reshape(-1))
    pos_pair = (position_table.reshape(2, HALF, D)
                .transpose(1, 0, 2).reshape(HALF, 2 * D))
    inter = _sc_gather(flat_ids, phoneme_table)
    inter = inter.reshape(b * t // 2, 2 * D)
    return _tc_add(inter, pos_pair, b)


# speedup vs baseline: 1.2438x; 1.2438x over previous
import functools

import jax
import jax.numpy as jnp
from jax import lax
from jax.experimental import pallas as pl
from jax.experimental.pallas import tpu as pltpu
from jax.experimental.pallas import tpu_sc as plsc

D = 64
T = 200
NC = 2
NS = 16
NW = NC * NS
C = 128
NBUF = 4
T_BLK = 8
B_BLK = 128


def _sc_gather(flat_ids, table):
    n_flat = flat_ids.shape[0]
    per_w = n_flat // NW
    n_chunks = per_w // C

    mesh = plsc.VectorSubcoreMesh(core_axis_name="c", subcore_axis_name="s")

    @functools.partial(
        pl.kernel,
        mesh=mesh,
        compiler_params=pltpu.CompilerParams(use_tc_tiling_on_sc=False),
        out_type=jax.ShapeDtypeStruct((n_flat, D), jnp.float32),
        scratch_types=[
            pltpu.VMEM((per_w,), jnp.int32),
        ] + [pltpu.VMEM((C, D), jnp.float32)] * NBUF
          + [pltpu.SemaphoreType.DMA] * (2 * NBUF),
    )
    def body(ids_hbm, table_hbm, out_hbm, idx_all,
             r0, r1, r2, r3, sg0, sg1, sg2, sg3, sw0, sw1, sw2, sw3):
        rows = (r0, r1, r2, r3)
        sg = (sg0, sg1, sg2, sg3)
        sw = (sw0, sw1, sw2, sw3)
        wid = lax.axis_index("s") * NC + lax.axis_index("c")
        base = wid * per_w
        pltpu.sync_copy(ids_hbm.at[pl.ds(base, per_w)], idx_all)

        def gather_start(g, b):
            pltpu.async_copy(table_hbm.at[idx_all.at[pl.ds(g * C, C)]],
                             rows[b], sg[b])

        def gather_wait(b):
            pltpu.make_async_copy(table_hbm.at[idx_all.at[pl.ds(0, C)]],
                                  rows[b], sg[b]).wait()

        def write_start(g, b):
            pltpu.async_copy(rows[b], out_hbm.at[pl.ds(base + g * C, C)],
                             sw[b])

        def write_wait(b):
            pltpu.make_async_copy(rows[b], out_hbm.at[pl.ds(base, C)],
                                  sw[b]).wait()

        for b in range(NBUF):
            gather_start(b, b)

        def main_body(i, carry):
            k = i * NBUF
            for b in range(NBUF):
                gather_wait(b)
                write_start(k + b, b)
            for b in range(NBUF):
                write_wait(b)
                gather_start(k + NBUF + b, b)
            return carry

        lax.fori_loop(0, n_chunks // NBUF - 1, main_body, 0)

        k = n_chunks - NBUF
        for b in range(NBUF):
            gather_wait(b)
            write_start(k + b, b)
        for b in range(NBUF):
            write_wait(b)

    return body(flat_ids, table)


def _tc_add_transpose(inter2, pos_b, batch):
    td = T * D

    def tc_body(inter_ref, pos_ref, out_ref):
        out_ref[...] = inter_ref[...].T + pos_ref[...]

    return pl.pallas_call(
        tc_body,
        grid=(T // T_BLK, batch // B_BLK),
        in_specs=[
            pl.BlockSpec((B_BLK, T_BLK * D), lambda it, ib: (ib, it)),
            pl.BlockSpec((T_BLK * D, B_BLK), lambda it, ib: (it, 0)),
        ],
        out_specs=pl.BlockSpec((T_BLK * D, B_BLK), lambda it, ib: (it, ib)),
        out_shape=jax.ShapeDtypeStruct((td, batch), jnp.float32),
    )(inter2, pos_b)


def kernel(input_ids, phoneme_table, position_table):
    b, t = input_ids.shape
    flat_ids = input_ids.reshape(-1).astype(jnp.int32)
    inter = _sc_gather(flat_ids, phoneme_table)
    inter2 = inter.reshape(b, t * D)
    pos_b = jnp.broadcast_to(position_table.reshape(-1)[:, None],
                             (t * D, B_BLK))
    out_td_b = _tc_add_transpose(inter2, pos_b, b)
    return out_td_b.reshape(t, D, b).transpose(2, 0, 1)


# speedup vs baseline: 1.8820x; 1.5131x over previous
import functools

import jax
import jax.numpy as jnp
from jax import lax
from jax.experimental import pallas as pl
from jax.experimental.pallas import tpu as pltpu
from jax.experimental.pallas import tpu_sc as plsc

D = 64
T = 200
NC = 2
NS = 16
NW = NC * NS
C = 128
NBUF = 4
T_BLK = 20
B_BLK = 256


def _sc_gather(flat_ids, table):
    n_flat = flat_ids.shape[0]
    per_w = n_flat // NW
    n_chunks = per_w // C

    mesh = plsc.VectorSubcoreMesh(core_axis_name="c", subcore_axis_name="s")

    @functools.partial(
        pl.kernel,
        mesh=mesh,
        compiler_params=pltpu.CompilerParams(use_tc_tiling_on_sc=False),
        out_type=jax.ShapeDtypeStruct((n_flat, D), jnp.float32),
        scratch_types=[
            pltpu.VMEM((per_w,), jnp.int32),
        ] + [pltpu.VMEM((C, D), jnp.float32)] * NBUF
          + [pltpu.SemaphoreType.DMA] * (2 * NBUF),
    )
    def body(ids_hbm, table_hbm, out_hbm, idx_all,
             r0, r1, r2, r3, sg0, sg1, sg2, sg3, sw0, sw1, sw2, sw3):
        rows = (r0, r1, r2, r3)
        sg = (sg0, sg1, sg2, sg3)
        sw = (sw0, sw1, sw2, sw3)
        wid = lax.axis_index("s") * NC + lax.axis_index("c")
        base = wid * per_w
        pltpu.sync_copy(ids_hbm.at[pl.ds(base, per_w)], idx_all)

        def gather_start(g, b):
            pltpu.async_copy(table_hbm.at[idx_all.at[pl.ds(g * C, C)]],
                             rows[b], sg[b])

        def gather_wait(b):
            pltpu.make_async_copy(table_hbm.at[idx_all.at[pl.ds(0, C)]],
                                  rows[b], sg[b]).wait()

        def write_start(g, b):
            pltpu.async_copy(rows[b], out_hbm.at[pl.ds(base + g * C, C)],
                             sw[b])

        def write_wait(b):
            pltpu.make_async_copy(rows[b], out_hbm.at[pl.ds(base, C)],
                                  sw[b]).wait()

        for b in range(NBUF):
            gather_start(b, b)

        def main_body(i, carry):
            k = i * NBUF
            for b in range(NBUF):
                gather_wait(b)
                write_start(k + b, b)
            for b in range(NBUF):
                write_wait(b)
                gather_start(k + NBUF + b, b)
            return carry

        lax.fori_loop(0, n_chunks // NBUF - 1, main_body, 0)

        k = n_chunks - NBUF
        for b in range(NBUF):
            gather_wait(b)
            write_start(k + b, b)
        for b in range(NBUF):
            write_wait(b)

    return body(flat_ids, table)


def _tc_add_transpose(inter2, pos_b, batch):
    td = T * D

    def tc_body(inter_ref, pos_ref, out_ref):
        out_ref[...] = inter_ref[...].T + pos_ref[...]

    return pl.pallas_call(
        tc_body,
        grid=(T // T_BLK, batch // B_BLK),
        in_specs=[
            pl.BlockSpec((B_BLK, T_BLK * D), lambda it, ib: (ib, it)),
            pl.BlockSpec((T_BLK * D, B_BLK), lambda it, ib: (it, 0)),
        ],
        out_specs=pl.BlockSpec((T_BLK * D, B_BLK), lambda it, ib: (it, ib)),
        out_shape=jax.ShapeDtypeStruct((td, batch), jnp.float32),
    )(inter2, pos_b)


def kernel(input_ids, phoneme_table, position_table):
    b, t = input_ids.shape
    flat_ids = input_ids.reshape(-1).astype(jnp.int32)
    inter = _sc_gather(flat_ids, phoneme_table)
    inter2 = inter.reshape(b, t * D)
    pos_b = jnp.broadcast_to(position_table.reshape(-1)[:, None],
                             (t * D, B_BLK))
    out_td_b = _tc_add_transpose(inter2, pos_b, b)
    return out_td_b.reshape(t, D, b).transpose(2, 0, 1)


# speedup vs baseline: 2.0559x; 1.0924x over previous
import functools

import jax
import jax.numpy as jnp
from jax import lax
from jax.experimental import pallas as pl
from jax.experimental.pallas import tpu as pltpu
from jax.experimental.pallas import tpu_sc as plsc

D = 64
T = 200
NC = 2
NS = 16
NW = NC * NS
C = 128
NBUF = 4
T_BLK = 40
B_BLK = 512


def _sc_gather(flat_ids, table):
    n_flat = flat_ids.shape[0]
    per_w = n_flat // NW
    n_chunks = per_w // C

    mesh = plsc.VectorSubcoreMesh(core_axis_name="c", subcore_axis_name="s")

    @functools.partial(
        pl.kernel,
        mesh=mesh,
        compiler_params=pltpu.CompilerParams(use_tc_tiling_on_sc=False),
        out_type=jax.ShapeDtypeStruct((n_flat, D), jnp.float32),
        scratch_types=[
            pltpu.VMEM((per_w,), jnp.int32),
        ] + [pltpu.VMEM((C, D), jnp.float32)] * NBUF
          + [pltpu.SemaphoreType.DMA] * (2 * NBUF),
    )
    def body(ids_hbm, table_hbm, out_hbm, idx_all,
             r0, r1, r2, r3, sg0, sg1, sg2, sg3, sw0, sw1, sw2, sw3):
        rows = (r0, r1, r2, r3)
        sg = (sg0, sg1, sg2, sg3)
        sw = (sw0, sw1, sw2, sw3)
        wid = lax.axis_index("s") * NC + lax.axis_index("c")
        base = wid * per_w
        pltpu.sync_copy(ids_hbm.at[pl.ds(base, per_w)], idx_all)

        def gather_start(g, b):
            pltpu.async_copy(table_hbm.at[idx_all.at[pl.ds(g * C, C)]],
                             rows[b], sg[b])

        def gather_wait(b):
            pltpu.make_async_copy(table_hbm.at[idx_all.at[pl.ds(0, C)]],
                                  rows[b], sg[b]).wait()

        def write_start(g, b):
            pltpu.async_copy(rows[b], out_hbm.at[pl.ds(base + g * C, C)],
                             sw[b])

        def write_wait(b):
            pltpu.make_async_copy(rows[b], out_hbm.at[pl.ds(base, C)],
                                  sw[b]).wait()

        for b in range(NBUF):
            gather_start(b, b)

        def main_body(i, carry):
            k = i * NBUF
            for b in range(NBUF):
                gather_wait(b)
                write_start(k + b, b)
            for b in range(NBUF):
                write_wait(b)
                gather_start(k + NBUF + b, b)
            return carry

        lax.fori_loop(0, n_chunks // NBUF - 1, main_body, 0)

        k = n_chunks - NBUF
        for b in range(NBUF):
            gather_wait(b)
            write_start(k + b, b)
        for b in range(NBUF):
            write_wait(b)

    return body(flat_ids, table)


def _tc_add_transpose(inter2, pos_b, batch):
    td = T * D

    def tc_body(inter_ref, pos_ref, out_ref):
        out_ref[...] = inter_ref[...].T + pos_ref[...]

    return pl.pallas_call(
        tc_body,
        grid=(T // T_BLK, batch // B_BLK),
        in_specs=[
            pl.BlockSpec((B_BLK, T_BLK * D), lambda it, ib: (ib, it)),
            pl.BlockSpec((T_BLK * D, B_BLK), lambda it, ib: (it, 0)),
        ],
        out_specs=pl.BlockSpec((T_BLK * D, B_BLK), lambda it, ib: (it, ib)),
        out_shape=jax.ShapeDtypeStruct((td, batch), jnp.float32),
    )(inter2, pos_b)


def kernel(input_ids, phoneme_table, position_table):
    b, t = input_ids.shape
    flat_ids = input_ids.reshape(-1).astype(jnp.int32)
    inter = _sc_gather(flat_ids, phoneme_table)
    inter2 = inter.reshape(b, t * D)
    pos_b = jnp.broadcast_to(position_table.reshape(-1)[:, None],
                             (t * D, B_BLK))
    out_td_b = _tc_add_transpose(inter2, pos_b, b)
    return out_td_b.reshape(t, D, b).transpose(2, 0, 1)


# speedup vs baseline: 2.8448x; 1.3837x over previous
import functools

import jax
import jax.numpy as jnp
from jax import lax
from jax.experimental import pallas as pl
from jax.experimental.pallas import tpu as pltpu
from jax.experimental.pallas import tpu_sc as plsc

D = 64
T = 200
NC = 2
NS = 16
NW = NC * NS
C = 128
NBUF = 4
T_BLK = 40
B_BLK = 512


def _sc_gather(flat_ids, table):
    n_flat = flat_ids.shape[0]
    per_w = n_flat // NW
    n_chunks = per_w // C

    mesh = plsc.VectorSubcoreMesh(core_axis_name="c", subcore_axis_name="s")

    @functools.partial(
        pl.kernel,
        mesh=mesh,
        compiler_params=pltpu.CompilerParams(use_tc_tiling_on_sc=False),
        out_type=jax.ShapeDtypeStruct((n_flat, D), jnp.float32),
        scratch_types=[
            pltpu.VMEM((per_w,), jnp.int32),
        ] + [pltpu.VMEM((C, D), jnp.float32)] * NBUF
          + [pltpu.SemaphoreType.DMA] * (2 * NBUF),
    )
    def body(ids_hbm, table_hbm, out_hbm, idx_all,
             r0, r1, r2, r3, sg0, sg1, sg2, sg3, sw0, sw1, sw2, sw3):
        rows = (r0, r1, r2, r3)
        sg = (sg0, sg1, sg2, sg3)
        sw = (sw0, sw1, sw2, sw3)
        wid = lax.axis_index("s") * NC + lax.axis_index("c")
        base = wid * per_w
        pltpu.sync_copy(ids_hbm.at[pl.ds(base, per_w)], idx_all)

        def gather_start(g, b):
            pltpu.async_copy(table_hbm.at[idx_all.at[pl.ds(g * C, C)]],
                             rows[b], sg[b])

        def gather_wait(b):
            pltpu.make_async_copy(table_hbm.at[idx_all.at[pl.ds(0, C)]],
                                  rows[b], sg[b]).wait()

        def write_start(g, b):
            pltpu.async_copy(rows[b], out_hbm.at[pl.ds(base + g * C, C)],
                             sw[b])

        def write_wait(b):
            pltpu.make_async_copy(rows[b], out_hbm.at[pl.ds(base, C)],
                                  sw[b]).wait()

        for b in range(NBUF):
            gather_start(b, b)

        def main_body(i, carry):
            k = i * NBUF
            for b in range(NBUF):
                gather_wait(b)
                write_start(k + b, b)
            for b in range(NBUF):
                write_wait(b)
                gather_start(k + NBUF + b, b)
            return carry

        lax.fori_loop(0, n_chunks // NBUF - 1, main_body, 0)

        k = n_chunks - NBUF
        for b in range(NBUF):
            gather_wait(b)
            write_start(k + b, b)
        for b in range(NBUF):
            write_wait(b)

    return body(flat_ids, table)


def _tc_add_transpose(inter_pair, pos_b, batch):
    td = T * D
    half = T // 2
    sblk = 128

    def tc_body(inter_ref, pos_ref, out_ref):
        x3 = inter_ref[...].reshape(sblk, half, 2 * D)
        y = jnp.transpose(x3, (1, 2, 0))
        out_ref[...] = y.reshape(td, sblk) + pos_ref[...]

    return pl.pallas_call(
        tc_body,
        grid=(batch // sblk,),
        in_specs=[
            pl.BlockSpec((sblk * half, 2 * D), lambda ib: (ib, 0)),
            pl.BlockSpec((td, sblk), lambda ib: (0, 0)),
        ],
        out_specs=pl.BlockSpec((td, sblk), lambda ib: (0, ib)),
        out_shape=jax.ShapeDtypeStruct((td, batch), jnp.float32),
    )(inter_pair, pos_b)


def kernel(input_ids, phoneme_table, position_table):
    b, t = input_ids.shape
    flat_ids = input_ids.reshape(-1).astype(jnp.int32)
    inter = _sc_gather(flat_ids, phoneme_table)
    inter_pair = inter.reshape(b * t // 2, 2 * D)
    pos_b = jnp.broadcast_to(position_table.reshape(-1)[:, None],
                             (t * D, 128))
    out_td_b = _tc_add_transpose(inter_pair, pos_b, b)
    return out_td_b.reshape(t, D, b).transpose(2, 0, 1)


# speedup vs baseline: 2.8515x; 1.0024x over previous
import functools

import jax
import jax.numpy as jnp
from jax import lax
from jax.experimental import pallas as pl
from jax.experimental.pallas import tpu as pltpu
from jax.experimental.pallas import tpu_sc as plsc

D = 64
T = 200
NC = 2
NS = 16
NW = NC * NS
C = 128
NBUF = 4
S_BLK = 128


def _sc_gather(flat_ids, table):
    n_flat = flat_ids.shape[0]
    per_w = n_flat // NW
    n_chunks = per_w // C

    mesh = plsc.VectorSubcoreMesh(core_axis_name="c", subcore_axis_name="s")

    @functools.partial(
        pl.kernel,
        mesh=mesh,
        compiler_params=pltpu.CompilerParams(use_tc_tiling_on_sc=False),
        out_type=jax.ShapeDtypeStruct((n_flat, D), jnp.float32),
        scratch_types=[
            pltpu.VMEM((per_w,), jnp.int32),
        ] + [pltpu.VMEM((C, D), jnp.float32)] * NBUF
          + [pltpu.SemaphoreType.DMA] * (2 * NBUF),
    )
    def body(ids_hbm, table_hbm, out_hbm, idx_all,
             r0, r1, r2, r3, sg0, sg1, sg2, sg3, sw0, sw1, sw2, sw3):
        rows = (r0, r1, r2, r3)
        sg = (sg0, sg1, sg2, sg3)
        sw = (sw0, sw1, sw2, sw3)
        wid = lax.axis_index("s") * NC + lax.axis_index("c")
        base = wid * per_w
        pltpu.sync_copy(ids_hbm.at[pl.ds(base, per_w)], idx_all)

        def gather_start(g, b):
            pltpu.async_copy(table_hbm.at[idx_all.at[pl.ds(g * C, C)]],
                             rows[b], sg[b])

        def gather_wait(b):
            pltpu.make_async_copy(table_hbm.at[idx_all.at[pl.ds(0, C)]],
                                  rows[b], sg[b]).wait()

        def write_start(g, b):
            pltpu.async_copy(rows[b], out_hbm.at[pl.ds(base + g * C, C)],
                             sw[b])

        def write_wait(b):
            pltpu.make_async_copy(rows[b], out_hbm.at[pl.ds(base, C)],
                                  sw[b]).wait()

        for b in range(NBUF):
            gather_start(b, b)

        def main_body(i, carry):
            k = i * NBUF
            for b in range(NBUF):
                gather_wait(b)
                write_start(k + b, b)
            for b in range(NBUF):
                write_wait(b)
                gather_start(k + NBUF + b, b)
            return carry

        lax.fori_loop(0, n_chunks // NBUF - 1, main_body, 0)

        k = n_chunks - NBUF
        for b in range(NBUF):
            gather_wait(b)
            write_start(k + b, b)
        for b in range(NBUF):
            write_wait(b)

    return body(flat_ids, table)


def _tc_add_transpose(inter_pair, pos_b, batch):
    td = T * D
    half = T // 2

    def tc_body(inter_ref, pos_ref, out_ref):
        x3 = inter_ref[...].reshape(S_BLK, half, 2 * D)
        y = jnp.transpose(x3, (1, 2, 0))
        out_ref[...] = y.reshape(td, S_BLK) + pos_ref[...]

    return pl.pallas_call(
        tc_body,
        grid=(batch // S_BLK,),
        in_specs=[
            pl.BlockSpec((S_BLK * half, 2 * D), lambda ib: (ib, 0)),
            pl.BlockSpec((td, S_BLK), lambda ib: (0, 0)),
        ],
        out_specs=pl.BlockSpec((td, S_BLK), lambda ib: (0, ib)),
        out_shape=jax.ShapeDtypeStruct((td, batch), jnp.float32),
    )(inter_pair, pos_b)


def kernel(input_ids, phoneme_table, position_table):
    b, t = input_ids.shape
    flat_ids = input_ids.reshape(-1).astype(jnp.int32)
    inter = _sc_gather(flat_ids, phoneme_table)
    inter_pair = inter.reshape(b * t // 2, 2 * D)
    pos_b = jnp.broadcast_to(position_table.reshape(-1)[:, None],
                             (t * D, S_BLK))
    out_td_b = _tc_add_transpose(inter_pair, pos_b, b)
    return out_td_b.reshape(t, D, b).transpose(2, 0, 1)
